# baseline (speedup 1.0000x reference)
import functools

import jax
import jax.numpy as jnp
from jax import lax
from jax.experimental import pallas as pl
from jax.experimental.pallas import tpu as pltpu
from jax.experimental.pallas import tpu_sc as plsc

VOCAB_ROWS = 1000000
BATCH = 1024
NUM_KW = 50
EMBED_DIM = 64
TOTAL = BATCH * NUM_KW
PACKED_ROWS = VOCAB_ROWS // 2
PACKED_DIM = 2 * EMBED_DIM

_info = plsc.get_sparse_core_info()
_NC, _NS = _info.num_cores, _info.num_subcores
_NW = _NC * _NS
_BPW = TOTAL // _NW
_LANES = _info.num_lanes

_mesh = plsc.VectorSubcoreMesh(core_axis_name="c", subcore_axis_name="s")


@functools.partial(
    pl.kernel,
    mesh=_mesh,
    out_type=jax.ShapeDtypeStruct((TOTAL, PACKED_DIM), jnp.bfloat16),
    scratch_types=[
        pltpu.VMEM((_BPW,), jnp.int32),
        pltpu.VMEM((_BPW, PACKED_DIM), jnp.bfloat16),
        pltpu.SemaphoreType.DMA,
    ],
    compiler_params=pltpu.CompilerParams(use_tc_tiling_on_sc=False),
)
def _gather_pairs(packed_hbm, idx_hbm, out_hbm, idx_v, rows_v, sem):
    wid = lax.axis_index("s") * _NC + lax.axis_index("c")
    base = wid * _BPW
    pltpu.sync_copy(idx_hbm.at[pl.ds(base, _BPW)], idx_v)
    for i in range(_BPW // _LANES):
        sl = pl.ds(i * _LANES, _LANES)
        idx_v[sl] = lax.shift_right_logical(idx_v[sl], 1)
    pltpu.async_copy(packed_hbm.at[idx_v], rows_v, sem).wait()
    pltpu.sync_copy(rows_v, out_hbm.at[pl.ds(base, _BPW)])


def kernel(kwids, kw_dist_adj, mask, word_embed_table):
    packed = (
        word_embed_table.T.astype(jnp.bfloat16).T.reshape(PACKED_ROWS, PACKED_DIM)
    )
    flat_ids = kwids.reshape(TOTAL)
    rows = _gather_pairs(packed, flat_ids).reshape(TOTAL, 2, EMBED_DIM)
    parity = (flat_ids & 1).astype(bool)
    kw_embed = (
        jnp.where(parity[:, None], rows[:, 1], rows[:, 0])
        .astype(jnp.float32)
        .reshape(BATCH, NUM_KW, EMBED_DIM)
    )
    return (kw_embed, kw_dist_adj, mask)


# speedup vs baseline: 1.2832x; 1.2832x over previous
import functools

import jax
import jax.numpy as jnp
from jax import lax
from jax.experimental import pallas as pl
from jax.experimental.pallas import tpu as pltpu
from jax.experimental.pallas import tpu_sc as plsc

VOCAB_ROWS = 1000000
BATCH = 1024
NUM_KW = 50
EMBED_DIM = 64
TOTAL = BATCH * NUM_KW
PACKED_ROWS = VOCAB_ROWS // 2
PACKED_DIM = 2 * EMBED_DIM

_info = plsc.get_sparse_core_info()
_NC, _NS = _info.num_cores, _info.num_subcores
_NW = _NC * _NS
_BPW = TOTAL // _NW
_CHUNK = _BPW // 2
_LANES = _info.num_lanes

_mesh = plsc.VectorSubcoreMesh(core_axis_name="c", subcore_axis_name="s")


@functools.partial(
    pl.kernel,
    mesh=_mesh,
    out_type=jax.ShapeDtypeStruct((TOTAL, PACKED_DIM), jnp.float32),
    scratch_types=[
        pltpu.VMEM((_BPW,), jnp.int32),
        pltpu.VMEM((_CHUNK, PACKED_DIM), jnp.float32),
        pltpu.SemaphoreType.DMA,
    ],
)
def _gather_pairs(packed_hbm, idx_hbm, out_hbm, idx_v, rows_v, sem):
    wid = lax.axis_index("s") * _NC + lax.axis_index("c")
    base = wid * _BPW
    pltpu.sync_copy(idx_hbm.at[pl.ds(base, _BPW)], idx_v)
    for i in range(_BPW // _LANES):
        sl = pl.ds(i * _LANES, _LANES)
        idx_v[sl] = lax.shift_right_logical(idx_v[sl], 1)
    for c in range(2):
        pltpu.async_copy(
            packed_hbm.at[idx_v.at[pl.ds(c * _CHUNK, _CHUNK)]], rows_v, sem
        ).wait()
        pltpu.sync_copy(rows_v, out_hbm.at[pl.ds(base + c * _CHUNK, _CHUNK)])


def kernel(kwids, kw_dist_adj, mask, word_embed_table):
    packed = word_embed_table.reshape(PACKED_ROWS, PACKED_DIM)
    flat_ids = kwids.reshape(TOTAL)
    rows = _gather_pairs(packed, flat_ids).reshape(TOTAL, 2, EMBED_DIM)
    parity = (flat_ids & 1).astype(bool)
    kw_embed = jnp.where(parity[:, None], rows[:, 1], rows[:, 0]).reshape(
        BATCH, NUM_KW, EMBED_DIM
    )
    return (kw_embed, kw_dist_adj, mask)


# speedup vs baseline: 1.5444x; 1.2035x over previous
import functools

import jax
import jax.numpy as jnp
from jax import lax
from jax.experimental import pallas as pl
from jax.experimental.pallas import tpu as pltpu
from jax.experimental.pallas import tpu_sc as plsc

VOCAB_ROWS = 1000000
BATCH = 1024
NUM_KW = 50
EMBED_DIM = 64
TOTAL = BATCH * NUM_KW
PACKED_ROWS = VOCAB_ROWS // 2
PACKED_DIM = 2 * EMBED_DIM

_TBLK = 2048
_TGRID = (VOCAB_ROWS + _TBLK - 1) // _TBLK
PACKED_PAD_ROWS = _TGRID * (_TBLK // 2)

_info = plsc.get_sparse_core_info()
_NC, _NS = _info.num_cores, _info.num_subcores
_NW = _NC * _NS
_BPW = TOTAL // _NW
_CHUNK = _BPW // 2
_LANES = _info.num_lanes

_mesh = plsc.VectorSubcoreMesh(core_axis_name="c", subcore_axis_name="s")


def _transpose_body(x_ref, o_ref):
    x = x_ref[...]
    eye = (
        lax.broadcasted_iota(jnp.int32, (EMBED_DIM, EMBED_DIM), 0)
        == lax.broadcasted_iota(jnp.int32, (EMBED_DIM, EMBED_DIM), 1)
    ).astype(jnp.float32)
    xt = lax.dot_general(
        x, eye, (((0,), (0,)), ((), ())), preferred_element_type=jnp.float32
    )
    o_ref[:, 0:EMBED_DIM] = xt[0 : _TBLK // 2, :]
    o_ref[:, EMBED_DIM:PACKED_DIM] = xt[_TBLK // 2 : _TBLK, :]


_transpose = pl.pallas_call(
    _transpose_body,
    grid=(_TGRID,),
    in_specs=[pl.BlockSpec((EMBED_DIM, _TBLK), lambda i: (0, i))],
    out_specs=pl.BlockSpec((_TBLK // 2, PACKED_DIM), lambda i: (i, 0)),
    out_shape=jax.ShapeDtypeStruct((PACKED_PAD_ROWS, PACKED_DIM), jnp.float32),
)


@functools.partial(
    pl.kernel,
    mesh=_mesh,
    out_type=jax.ShapeDtypeStruct((TOTAL, PACKED_DIM), jnp.float32),
    scratch_types=[
        pltpu.VMEM((_BPW,), jnp.int32),
        pltpu.VMEM((_CHUNK, PACKED_DIM), jnp.float32),
        pltpu.SemaphoreType.DMA,
    ],
)
def _gather_pairs(packed_hbm, idx_hbm, out_hbm, idx_v, rows_v, sem):
    wid = lax.axis_index("s") * _NC + lax.axis_index("c")
    base = wid * _BPW
    pltpu.sync_copy(idx_hbm.at[pl.ds(base, _BPW)], idx_v)
    for i in range(_BPW // _LANES):
        sl = pl.ds(i * _LANES, _LANES)
        v = idx_v[sl]
        idx_v[sl] = lax.shift_left(lax.shift_right_logical(v, 11), 10) | (v & 1023)
    for c in range(2):
        pltpu.async_copy(
            packed_hbm.at[idx_v.at[pl.ds(c * _CHUNK, _CHUNK)]], rows_v, sem
        ).wait()
        pltpu.sync_copy(rows_v, out_hbm.at[pl.ds(base + c * _CHUNK, _CHUNK)])


def kernel(kwids, kw_dist_adj, mask, word_embed_table):
    packed = _transpose(word_embed_table.T)
    flat_ids = kwids.reshape(TOTAL)
    rows = _gather_pairs(packed, flat_ids).reshape(TOTAL, 2, EMBED_DIM)
    upper = ((flat_ids >> 10) & 1).astype(bool)
    kw_embed = jnp.where(upper[:, None], rows[:, 1], rows[:, 0]).reshape(
        BATCH, NUM_KW, EMBED_DIM
    )
    return (kw_embed, kw_dist_adj, mask)


# speedup vs baseline: 1.8414x; 1.1923x over previous
import functools

import jax
import jax.numpy as jnp
from jax import lax
from jax.experimental import pallas as pl
from jax.experimental.pallas import tpu as pltpu
from jax.experimental.pallas import tpu_sc as plsc

VOCAB_ROWS = 1000000
BATCH = 1024
NUM_KW = 50
EMBED_DIM = 64
TOTAL = BATCH * NUM_KW
PACKED_ROWS = VOCAB_ROWS // 2
PACKED_DIM = 2 * EMBED_DIM

_TBLK = 2048
_TGRID = (VOCAB_ROWS + _TBLK - 1) // _TBLK
PACKED_PAD_ROWS = _TGRID * (_TBLK // 2)

_info = plsc.get_sparse_core_info()
_NC, _NS = _info.num_cores, _info.num_subcores
_NW = _NC * _NS
_BPW = TOTAL // _NW
_CHUNK = _BPW // 2
_LANES = _info.num_lanes

_mesh = plsc.VectorSubcoreMesh(core_axis_name="c", subcore_axis_name="s")


def _transpose_body(x_ref, o_ref):
    x = x_ref[...]
    eye = (
        lax.broadcasted_iota(jnp.int32, (EMBED_DIM, EMBED_DIM), 0)
        == lax.broadcasted_iota(jnp.int32, (EMBED_DIM, EMBED_DIM), 1)
    ).astype(jnp.float32)
    xt = lax.dot_general(
        x, eye, (((0,), (0,)), ((), ())), preferred_element_type=jnp.float32
    )
    o_ref[:, 0:EMBED_DIM] = xt[0 : _TBLK // 2, :]
    o_ref[:, EMBED_DIM:PACKED_DIM] = xt[_TBLK // 2 : _TBLK, :]


_transpose = pl.pallas_call(
    _transpose_body,
    grid=(_TGRID,),
    in_specs=[pl.BlockSpec((EMBED_DIM, _TBLK), lambda i: (0, i))],
    out_specs=pl.BlockSpec((_TBLK // 2, PACKED_DIM), lambda i: (i, 0)),
    out_shape=jax.ShapeDtypeStruct((PACKED_PAD_ROWS, PACKED_DIM), jnp.float32),
)


@functools.partial(
    pl.kernel,
    mesh=_mesh,
    out_type=jax.ShapeDtypeStruct((TOTAL, PACKED_DIM), jnp.float32),
    scratch_types=[
        pltpu.VMEM((_BPW,), jnp.int32),
        pltpu.VMEM((_CHUNK, PACKED_DIM), jnp.float32),
        pltpu.SemaphoreType.DMA,
    ],
)
def _gather_pairs(packed_hbm, idx_hbm, out_hbm, idx_v, rows_v, sem):
    wid = lax.axis_index("s") * _NC + lax.axis_index("c")
    base = wid * _BPW
    pltpu.sync_copy(idx_hbm.at[pl.ds(base, _BPW)], idx_v)
    for i in range(_BPW // _LANES):
        sl = pl.ds(i * _LANES, _LANES)
        v = idx_v[sl]
        idx_v[sl] = lax.shift_left(lax.shift_right_logical(v, 11), 10) | (v & 1023)
    for c in range(2):
        pltpu.async_copy(
            packed_hbm.at[idx_v.at[pl.ds(c * _CHUNK, _CHUNK)]], rows_v, sem
        ).wait()
        pltpu.sync_copy(rows_v, out_hbm.at[pl.ds(base + c * _CHUNK, _CHUNK)])


def kernel(kwids, kw_dist_adj, mask, word_embed_table):
    packed = _transpose(word_embed_table.T)
    flat_ids = kwids.reshape(TOTAL)
    rows = _gather_pairs(packed, flat_ids)
    upper = ((flat_ids >> 10) & 1).astype(bool)
    kw_embed = jnp.where(
        upper[:, None], rows[:, EMBED_DIM:PACKED_DIM], rows[:, 0:EMBED_DIM]
    ).reshape(BATCH, NUM_KW, EMBED_DIM)
    return (kw_embed, kw_dist_adj, mask)


# speedup vs baseline: 2.3835x; 1.2944x over previous
import functools

import jax
import jax.numpy as jnp
from jax import lax
from jax.experimental import pallas as pl
from jax.experimental.pallas import tpu as pltpu
from jax.experimental.pallas import tpu_sc as plsc

VOCAB_ROWS = 1000000
BATCH = 1024
NUM_KW = 50
EMBED_DIM = 64
TOTAL = BATCH * NUM_KW
PACKED_ROWS = VOCAB_ROWS // 2
PACKED_DIM = 2 * EMBED_DIM

_TBLK = 4096
_TSHIFT = _TBLK.bit_length() - 1
_THALF = _TBLK // 2
_TGRID = (VOCAB_ROWS + _TBLK - 1) // _TBLK
PACKED_PAD_ROWS = _TGRID * _THALF

_info = plsc.get_sparse_core_info()
_NC, _NS = _info.num_cores, _info.num_subcores
_NW = _NC * _NS
_BPW = TOTAL // _NW
_CHUNK = _BPW // 2
_LANES = _info.num_lanes

_mesh = plsc.VectorSubcoreMesh(core_axis_name="c", subcore_axis_name="s")


def _transpose_body(x_ref, o_ref):
    x = x_ref[...]
    eye = (
        lax.broadcasted_iota(jnp.int32, (EMBED_DIM, EMBED_DIM), 0)
        == lax.broadcasted_iota(jnp.int32, (EMBED_DIM, EMBED_DIM), 1)
    ).astype(jnp.float32)
    xt = lax.dot_general(
        x, eye, (((0,), (0,)), ((), ())), preferred_element_type=jnp.float32
    )
    o_ref[:, 0:EMBED_DIM] = xt[0:_THALF, :]
    o_ref[:, EMBED_DIM:PACKED_DIM] = xt[_THALF:_TBLK, :]


_transpose = pl.pallas_call(
    _transpose_body,
    grid=(_TGRID,),
    in_specs=[pl.BlockSpec((EMBED_DIM, _TBLK), lambda i: (0, i))],
    out_specs=pl.BlockSpec((_THALF, PACKED_DIM), lambda i: (i, 0)),
    out_shape=jax.ShapeDtypeStruct((PACKED_PAD_ROWS, PACKED_DIM), jnp.float32),
)


@functools.partial(
    pl.kernel,
    mesh=_mesh,
    out_type=jax.ShapeDtypeStruct((TOTAL, PACKED_DIM), jnp.float32),
    scratch_types=[
        pltpu.VMEM((_BPW,), jnp.int32),
        pltpu.VMEM((_CHUNK, PACKED_DIM), jnp.float32),
        pltpu.SemaphoreType.DMA,
    ],
)
def _gather_pairs(packed_hbm, idx_hbm, out_hbm, idx_v, rows_v, sem):
    wid = lax.axis_index("s") * _NC + lax.axis_index("c")
    base = wid * _BPW
    pltpu.sync_copy(idx_hbm.at[pl.ds(base, _BPW)], idx_v)
    for i in range(_BPW // _LANES):
        sl = pl.ds(i * _LANES, _LANES)
        v = idx_v[sl]
        idx_v[sl] = lax.shift_left(
            lax.shift_right_logical(v, _TSHIFT), _TSHIFT - 1
        ) | (v & (_THALF - 1))
    for c in range(2):
        pltpu.async_copy(
            packed_hbm.at[idx_v.at[pl.ds(c * _CHUNK, _CHUNK)]], rows_v, sem
        ).wait()
        pltpu.sync_copy(rows_v, out_hbm.at[pl.ds(base + c * _CHUNK, _CHUNK)])


def kernel(kwids, kw_dist_adj, mask, word_embed_table):
    packed = _transpose(word_embed_table.T)
    flat_ids = kwids.reshape(TOTAL)
    rows = _gather_pairs(packed, flat_ids)
    upper = ((flat_ids >> (_TSHIFT - 1)) & 1).astype(bool)
    kw_embed = jnp.where(
        upper[:, None], rows[:, EMBED_DIM:PACKED_DIM], rows[:, 0:EMBED_DIM]
    ).reshape(BATCH, NUM_KW, EMBED_DIM)
    return (kw_embed, kw_dist_adj, mask)


# speedup vs baseline: 2.8075x; 1.1779x over previous
import functools

import jax
import jax.numpy as jnp
from jax import lax
from jax.experimental import pallas as pl
from jax.experimental.pallas import tpu as pltpu
from jax.experimental.pallas import tpu_sc as plsc

VOCAB_ROWS = 1000000
BATCH = 1024
NUM_KW = 50
EMBED_DIM = 64
TOTAL = BATCH * NUM_KW
PACKED_ROWS = VOCAB_ROWS // 2
PACKED_DIM = 2 * EMBED_DIM

_TBLK = 8192
_TSHIFT = _TBLK.bit_length() - 1
_THALF = _TBLK // 2
_TGRID = (VOCAB_ROWS + _TBLK - 1) // _TBLK
PACKED_PAD_ROWS = _TGRID * _THALF

_info = plsc.get_sparse_core_info()
_NC, _NS = _info.num_cores, _info.num_subcores
_NW = _NC * _NS
_BPW = TOTAL // _NW
_CHUNK = _BPW // 2
_LANES = _info.num_lanes

_mesh = plsc.VectorSubcoreMesh(core_axis_name="c", subcore_axis_name="s")


def _transpose_body(x_ref, o_ref):
    x = x_ref[...]
    eye = (
        lax.broadcasted_iota(jnp.int32, (EMBED_DIM, EMBED_DIM), 0)
        == lax.broadcasted_iota(jnp.int32, (EMBED_DIM, EMBED_DIM), 1)
    ).astype(jnp.float32)
    xt = lax.dot_general(
        x, eye, (((0,), (0,)), ((), ())), preferred_element_type=jnp.float32
    )
    o_ref[:, 0:EMBED_DIM] = xt[0:_THALF, :]
    o_ref[:, EMBED_DIM:PACKED_DIM] = xt[_THALF:_TBLK, :]


_transpose = pl.pallas_call(
    _transpose_body,
    grid=(_TGRID,),
    in_specs=[pl.BlockSpec((EMBED_DIM, _TBLK), lambda i: (0, i))],
    out_specs=pl.BlockSpec((_THALF, PACKED_DIM), lambda i: (i, 0)),
    out_shape=jax.ShapeDtypeStruct((PACKED_PAD_ROWS, PACKED_DIM), jnp.float32),
)


@functools.partial(
    pl.kernel,
    mesh=_mesh,
    out_type=jax.ShapeDtypeStruct((TOTAL, PACKED_DIM), jnp.float32),
    scratch_types=[
        pltpu.VMEM((_BPW,), jnp.int32),
        pltpu.VMEM((_CHUNK, PACKED_DIM), jnp.float32),
        pltpu.SemaphoreType.DMA,
    ],
)
def _gather_pairs(packed_hbm, idx_hbm, out_hbm, idx_v, rows_v, sem):
    wid = lax.axis_index("s") * _NC + lax.axis_index("c")
    base = wid * _BPW
    pltpu.sync_copy(idx_hbm.at[pl.ds(base, _BPW)], idx_v)
    for i in range(_BPW // _LANES):
        sl = pl.ds(i * _LANES, _LANES)
        v = idx_v[sl]
        idx_v[sl] = lax.shift_left(
            lax.shift_right_logical(v, _TSHIFT), _TSHIFT - 1
        ) | (v & (_THALF - 1))
    for c in range(2):
        pltpu.async_copy(
            packed_hbm.at[idx_v.at[pl.ds(c * _CHUNK, _CHUNK)]], rows_v, sem
        ).wait()
        pltpu.sync_copy(rows_v, out_hbm.at[pl.ds(base + c * _CHUNK, _CHUNK)])


def kernel(kwids, kw_dist_adj, mask, word_embed_table):
    packed = _transpose(word_embed_table.T)
    flat_ids = kwids.reshape(TOTAL)
    rows = _gather_pairs(packed, flat_ids)
    upper = ((flat_ids >> (_TSHIFT - 1)) & 1).astype(bool)
    kw_embed = jnp.where(
        upper[:, None], rows[:, EMBED_DIM:PACKED_DIM], rows[:, 0:EMBED_DIM]
    ).reshape(BATCH, NUM_KW, EMBED_DIM)
    return (kw_embed, kw_dist_adj, mask)
